# baseline (speedup 1.0000x reference)
import functools

import jax
import jax.numpy as jnp
from jax import lax
from jax.experimental import pallas as pl
from jax.experimental.pallas import tpu as pltpu
from jax.experimental.pallas import tpu_sc as plsc

V = 1000
D = 128
B = 4096 * 50
NC, NS = 2, 16
NW = NC * NS
B_PER_W = B // NW
CHUNK = 400
NCHUNK = B_PER_W // CHUNK
NBUF = 2
NOUTER = NCHUNK // NBUF

_mesh = plsc.VectorSubcoreMesh(core_axis_name="c", subcore_axis_name="s")


@functools.partial(
    pl.kernel,
    mesh=_mesh,
    out_type=jax.ShapeDtypeStruct((B, D), jnp.float32),
    scratch_types=[
        pltpu.VMEM((B_PER_W,), jnp.int32),
        pltpu.VMEM((NBUF, CHUNK, D), jnp.float32),
        pltpu.SemaphoreType.DMA((NBUF,)),
        pltpu.SemaphoreType.DMA((NBUF,)),
    ],
)
def _gather_kernel(idx_hbm, table_hbm, out_hbm, idx_v, rows_v, gsem, osem):
    wid = lax.axis_index("s") * NC + lax.axis_index("c")
    base = wid * B_PER_W
    pltpu.sync_copy(idx_hbm.at[pl.ds(base, B_PER_W)], idx_v)

    def gather(b, off):
        return pltpu.make_async_copy(
            table_hbm.at[idx_v.at[pl.ds(off, CHUNK)]], rows_v.at[b], gsem.at[b]
        )

    def out_copy(b, off):
        return pltpu.make_async_copy(
            rows_v.at[b], out_hbm.at[pl.ds(base + off, CHUNK)], osem.at[b]
        )

    for b in range(NBUF):
        gather(b, b * CHUNK).start()

    def body(g, carry):
        for b in range(NBUF):
            off = (g * NBUF + b) * CHUNK
            gather(b, off).wait()
            out_copy(b, off).start()
            out_copy(b, off).wait()
            gather(b, off + NBUF * CHUNK).start()
        return carry

    lax.fori_loop(0, NOUTER - 1, body, 0, unroll=False)

    for b in range(NBUF):
        off = ((NOUTER - 1) * NBUF + b) * CHUNK
        gather(b, off).wait()
        out_copy(b, off).start()
        out_copy(b, off).wait()


def kernel(prefix_token_ids, prefix_embedding):
    idx = prefix_token_ids.reshape(-1).astype(jnp.int32)
    out = _gather_kernel(idx, prefix_embedding)
    return out.reshape(prefix_token_ids.shape + (D,))


# speedup vs baseline: 1.6221x; 1.6221x over previous
import functools

import jax
import jax.numpy as jnp
from jax import lax
from jax.experimental import pallas as pl
from jax.experimental.pallas import tpu as pltpu
from jax.experimental.pallas import tpu_sc as plsc

NB = 4096
T = 50
D = 128
NC, NS = 2, 16
NW = NC * NS
ROWS_PER_W = NB // NW
IDX_PER_W = ROWS_PER_W * T
CHUNK_ROWS = 8
CHUNK = CHUNK_ROWS * T
NCHUNK = ROWS_PER_W // CHUNK_ROWS
NBUF = 2
NOUTER = NCHUNK // NBUF

_mesh = plsc.VectorSubcoreMesh(core_axis_name="c", subcore_axis_name="s")


@functools.partial(
    pl.kernel,
    mesh=_mesh,
    out_type=jax.ShapeDtypeStruct((NB, T, D), jnp.float32),
    scratch_types=[
        pltpu.VMEM((IDX_PER_W,), jnp.int32),
        pltpu.VMEM((NBUF, CHUNK, D), jnp.float32),
        pltpu.SemaphoreType.DMA((NBUF,)),
        pltpu.SemaphoreType.DMA((NBUF,)),
    ],
)
def _gather_kernel(idx_hbm, table_hbm, out_hbm, idx_v, rows_v, gsem, osem):
    wid = lax.axis_index("s") * NC + lax.axis_index("c")
    row_base = wid * ROWS_PER_W
    pltpu.sync_copy(idx_hbm.at[pl.ds(wid * IDX_PER_W, IDX_PER_W)], idx_v)

    def gather(b, c):
        return pltpu.make_async_copy(
            table_hbm.at[idx_v.at[pl.ds(c * CHUNK, CHUNK)]],
            rows_v.at[b],
            gsem.at[b],
        )

    def out_copies(b, c):
        return [
            pltpu.make_async_copy(
                rows_v.at[b].at[pl.ds(j * T, T)],
                out_hbm.at[row_base + c * CHUNK_ROWS + j],
                osem.at[b],
            )
            for j in range(CHUNK_ROWS)
        ]

    for b in range(NBUF):
        gather(b, b).start()

    def body(g, carry):
        for b in range(NBUF):
            c = g * NBUF + b
            gather(b, c).wait()
            cps = out_copies(b, c)
            for cp in cps:
                cp.start()
            for cp in cps:
                cp.wait()
            gather(b, c + NBUF).start()
        return carry

    lax.fori_loop(0, NOUTER - 1, body, 0, unroll=False)

    for b in range(NBUF):
        c = (NOUTER - 1) * NBUF + b
        gather(b, c).wait()
        cps = out_copies(b, c)
        for cp in cps:
            cp.start()
        for cp in cps:
            cp.wait()


def kernel(prefix_token_ids, prefix_embedding):
    idx = prefix_token_ids.reshape(-1).astype(jnp.int32)
    return _gather_kernel(idx, prefix_embedding)


# speedup vs baseline: 2.5148x; 1.5504x over previous
import functools

import jax
import jax.numpy as jnp
from jax import lax
from jax.experimental import pallas as pl
from jax.experimental.pallas import tpu as pltpu
from jax.experimental.pallas import tpu_sc as plsc

NB = 4096
T = 50
D = 128
NC, NS = 2, 16
NW = NC * NS
BPW = NB // NW
NBUF = 5
NOUTER = T // NBUF

_mesh = plsc.VectorSubcoreMesh(core_axis_name="c", subcore_axis_name="s")


@functools.partial(
    pl.kernel,
    mesh=_mesh,
    out_type=jax.ShapeDtypeStruct((T, NB, D), jnp.float32),
    scratch_types=[
        pltpu.VMEM((T, BPW), jnp.int32),
        pltpu.VMEM((NBUF, BPW, D), jnp.float32),
        pltpu.SemaphoreType.DMA((NBUF,)),
        pltpu.SemaphoreType.DMA((NBUF,)),
    ],
)
def _gather_kernel(idx_hbm, table_hbm, out_hbm, idx_v, rows_v, gsem, osem):
    wid = lax.axis_index("s") * NC + lax.axis_index("c")
    col0 = wid * BPW
    pltpu.sync_copy(idx_hbm.at[:, pl.ds(col0, BPW)], idx_v)

    def gather(b, t):
        return pltpu.make_async_copy(
            table_hbm.at[idx_v.at[t]], rows_v.at[b], gsem.at[b]
        )

    def out_copy(b, t):
        return pltpu.make_async_copy(
            rows_v.at[b], out_hbm.at[t, pl.ds(col0, BPW)], osem.at[b]
        )

    for b in range(NBUF):
        gather(b, b).start()

    def body(g, carry):
        for b in range(NBUF):
            t = g * NBUF + b
            gather(b, t).wait()
            out_copy(b, t).start()
            out_copy(b, t).wait()
            gather(b, t + NBUF).start()
        return carry

    lax.fori_loop(0, NOUTER - 1, body, 0, unroll=False)

    for b in range(NBUF):
        t = (NOUTER - 1) * NBUF + b
        gather(b, t).wait()
        out_copy(b, t).start()
        out_copy(b, t).wait()


def kernel(prefix_token_ids, prefix_embedding):
    idx_t = prefix_token_ids.T.astype(jnp.int32)
    out_t = _gather_kernel(idx_t, prefix_embedding)
    return out_t.transpose(1, 0, 2)


# speedup vs baseline: 5.4951x; 2.1851x over previous
import functools

import jax
import jax.numpy as jnp
from jax import lax
from jax.experimental import pallas as pl
from jax.experimental.pallas import tpu as pltpu
from jax.experimental.pallas import tpu_sc as plsc

NB = 4096
T = 50
D = 128
NC, NS = 2, 16
NW = NC * NS
BPW = NB // NW
NBUF = 5
NOUTER = T // NBUF

_mesh = plsc.VectorSubcoreMesh(core_axis_name="c", subcore_axis_name="s")


@functools.partial(
    pl.kernel,
    mesh=_mesh,
    out_type=jax.ShapeDtypeStruct((T, NB, D), jnp.float32),
    scratch_types=[
        pltpu.VMEM((T, BPW), jnp.int32),
        pltpu.VMEM((NBUF, BPW, D), jnp.float32),
        pltpu.VMEM_SHARED((1000, D), jnp.float32),
        pltpu.SemaphoreType.DMA((NBUF,)),
        pltpu.SemaphoreType.DMA((NBUF,)),
    ],
)
def _gather_kernel(idx_hbm, table_hbm, out_hbm, idx_v, rows_v, table_sp, gsem, osem):
    wid = lax.axis_index("s") * NC + lax.axis_index("c")
    col0 = wid * BPW

    @pl.when(lax.axis_index("s") == 0)
    def _():
        pltpu.sync_copy(table_hbm, table_sp)

    pltpu.sync_copy(idx_hbm.at[:, pl.ds(col0, BPW)], idx_v)
    plsc.subcore_barrier()

    def gather(b, t):
        return pltpu.make_async_copy(
            table_sp.at[idx_v.at[t]], rows_v.at[b], gsem.at[b]
        )

    def out_copy(b, t):
        return pltpu.make_async_copy(
            rows_v.at[b], out_hbm.at[t, pl.ds(col0, BPW)], osem.at[b]
        )

    for b in range(NBUF):
        gather(b, b).start()

    def body(g, carry):
        for b in range(NBUF):
            t = g * NBUF + b
            gather(b, t).wait()
            out_copy(b, t).start()
            out_copy(b, t).wait()
            gather(b, t + NBUF).start()
        return carry

    lax.fori_loop(0, NOUTER - 1, body, 0, unroll=False)

    for b in range(NBUF):
        t = (NOUTER - 1) * NBUF + b
        gather(b, t).wait()
        out_copy(b, t).start()
        out_copy(b, t).wait()


def kernel(prefix_token_ids, prefix_embedding):
    idx_t = prefix_token_ids.T.astype(jnp.int32)
    out_t = _gather_kernel(idx_t, prefix_embedding)
    return out_t.transpose(1, 0, 2)
